# baseline (speedup 1.0000x reference)
import functools

import jax
import jax.numpy as jnp
from jax import lax
from jax.experimental import pallas as pl
from jax.experimental.pallas import tpu as pltpu
from jax.experimental.pallas import tpu_sc as plsc

NC = 2
NS = 16
NW = NC * NS

CHUNK = 256
SUB = CHUNK // 128
FGRP = 16


def _sc_partial_sums(x, batch2d, n_rows, d, g):
    num_chunks = n_rows // CHUNK
    k_max = -(-num_chunks // NW)
    if k_max % 2:
        k_max += 1
    tail_n = num_chunks - NW * (k_max - 1)
    pairs = k_max // 2
    mesh = plsc.VectorSubcoreMesh(
        core_axis_name="c", subcore_axis_name="s", num_cores=NC, num_subcores=NS
    )
    rows_per_tile = g // NS
    zrows = 8
    nk = d // 16

    @functools.partial(
        pl.kernel,
        out_type=jax.ShapeDtypeStruct((NC, g, d), jnp.float32),
        mesh=mesh,
        scratch_types=[
            pltpu.VMEM((2, SUB, 128), jnp.int32),
            pltpu.VMEM((2, CHUNK, d), jnp.float32),
            pltpu.VMEM((CHUNK, d), jnp.float32),
            pltpu.VMEM((CHUNK // FGRP, FGRP), jnp.int32),
            pltpu.VMEM((zrows, d), jnp.float32),
            pltpu.VMEM_SHARED((g + FGRP, d), jnp.float32),
            pltpu.SemaphoreType.DMA,
            pltpu.SemaphoreType.DMA,
        ],
    )
    def sc_kernel(
        x_hbm, b_hbm, out_hbm, ids_v, rows_v, fbuf_v, fids_v, zbuf_v, acc,
        sem0, sem1,
    ):
        cid = lax.axis_index("c")
        sid = lax.axis_index("s")
        wid = sid * NC + cid
        sems = (sem0, sem1)

        zeros16 = jnp.zeros((16,), jnp.float32)
        dump16 = jnp.full((FGRP,), g, jnp.int32)
        lane = lax.iota(jnp.int32, 16)
        lane0 = lane == 0

        @pl.loop(0, zrows)
        def _zero(i):
            for k in range(nk):
                zbuf_v[i, pl.ds(k * 16, 16)] = zeros16

        for i in range(rows_per_tile // zrows):
            pltpu.sync_copy(
                zbuf_v, acc.at[pl.ds(sid * rows_per_tile + i * zrows, zrows)]
            )
        plsc.subcore_barrier()

        def start(k, b):
            j = wid + NW * k
            pltpu.async_copy(
                x_hbm.at[pl.ds(j * CHUNK, CHUNK)], rows_v.at[b], sems[b]
            )
            pltpu.async_copy(
                b_hbm.at[pl.ds(j * SUB, SUB)], ids_v.at[b], sems[b]
            )

        def wait(b):
            pltpu.make_async_copy(
                x_hbm.at[pl.ds(0, CHUNK)], rows_v.at[b], sems[b]
            ).wait()
            pltpu.make_async_copy(
                b_hbm.at[pl.ds(0, SUB)], ids_v.at[b], sems[b]
            ).wait()

        def process_chunk(b):
            for gi in range(CHUNK // FGRP):
                fids_v[gi, :] = dump16

            id0 = ids_v[b, 0, pl.ds(0, 16)][0]

            def group(gi, carry):
                prev_id, cnt = carry[0], carry[1]
                a = list(carry[2:])
                idvec = ids_v[b, gi // 8, pl.ds((gi % 8) * 16, 16)]
                for l in range(16):
                    id_r = idvec[l]
                    bnd = id_r != prev_id
                    cnt = cnt + bnd.astype(jnp.int32)
                    r = gi * 16 + l
                    for k in range(nk):
                        v = rows_v[b, r, pl.ds(k * 16, 16)]
                        a[k] = jnp.where(bnd, v, a[k] + v)
                        fbuf_v[cnt, pl.ds(k * 16, 16)] = a[k]
                    frow = fids_v[cnt // FGRP, pl.ds(0, FGRP)]
                    fids_v[cnt // FGRP, pl.ds(0, FGRP)] = jnp.where(
                        lane == cnt % FGRP, id_r, frow
                    )
                    prev_id = id_r
                return tuple([prev_id, cnt] + a)

            carry = lax.fori_loop(
                0,
                CHUNK // 16,
                group,
                tuple([id0, jnp.int32(0)] + [zeros16] * nk),
            )
            ngrp = (carry[1] + FGRP) // FGRP

            @pl.loop(0, ngrp)
            def _flush(gi):
                pltpu.sync_copy(
                    fbuf_v.at[pl.ds(gi * FGRP, FGRP)],
                    acc.at[fids_v.at[gi]],
                    add=True,
                )

        start(0, 0)

        @pl.loop(0, pairs)
        def _ring(p):
            last_a = p == pairs - 1

            @pl.when(jnp.logical_or(~last_a, wid < tail_n))
            def _():
                start(2 * p + 1, 1)

            wait(0)
            process_chunk(0)

            @pl.when(~last_a)
            def _():
                start(2 * p + 2, 0)

            @pl.when(jnp.logical_or(~last_a, wid < tail_n))
            def _():
                wait(1)
                process_chunk(1)

        plsc.subcore_barrier()
        pltpu.sync_copy(
            acc.at[pl.ds(sid * rows_per_tile, rows_per_tile)],
            out_hbm.at[cid, pl.ds(sid * rows_per_tile, rows_per_tile)],
        )

    return sc_kernel(x, batch2d)


def _combine_body(p_ref, o_ref):
    o_ref[...] = p_ref[0] + p_ref[1]


def kernel(x, batch, ptr):
    n, d = x.shape
    g = int(ptr.shape[0]) - 1
    batch2d = batch.astype(jnp.int32).reshape(n // 128, 128)
    partials = _sc_partial_sums(x, batch2d, n, d, g)
    out = pl.pallas_call(
        _combine_body,
        out_shape=jax.ShapeDtypeStruct((g, d), jnp.float32),
    )(partials)
    return out


# speedup vs baseline: 4.5015x; 4.5015x over previous
import functools

import jax
import jax.numpy as jnp
from jax import lax
from jax.experimental import pallas as pl
from jax.experimental.pallas import tpu as pltpu
from jax.experimental.pallas import tpu_sc as plsc

NC = 2
NS = 16
NW = NC * NS

SB = 128
DEPTH = 4


def _sc_partial_sums(x, batch2d, n_rows, d, g):
    num_sb = n_rows // SB
    unif = num_sb // NW
    tail_n = num_sb - NW * unif
    mesh = plsc.VectorSubcoreMesh(
        core_axis_name="c", subcore_axis_name="s", num_cores=NC, num_subcores=NS
    )
    rows_per_tile = g // NS
    zrows = 8

    @functools.partial(
        pl.kernel,
        out_type=jax.ShapeDtypeStruct((NC, g, d), jnp.float32),
        mesh=mesh,
        scratch_types=[
            pltpu.VMEM((DEPTH, 1, 128), jnp.int32),
            pltpu.VMEM((DEPTH, SB, d), jnp.float32),
            pltpu.VMEM((zrows, d), jnp.float32),
            pltpu.VMEM_SHARED((g, d), jnp.float32),
        ]
        + [pltpu.SemaphoreType.DMA] * (2 * DEPTH),
    )
    def sc_kernel(x_hbm, b_hbm, out_hbm, ids_v, rows_v, zbuf_v, acc, *sems_all):
        cid = lax.axis_index("c")
        sid = lax.axis_index("s")
        wid = sid * NC + cid
        sems = sems_all[:DEPTH]
        ssems = sems_all[DEPTH:]

        zeros16 = jnp.zeros((16,), jnp.float32)

        @pl.loop(0, zrows)
        def _zero(i):
            for k in range(d // 16):
                zbuf_v[i, pl.ds(k * 16, 16)] = zeros16

        for i in range(rows_per_tile // zrows):
            pltpu.sync_copy(
                zbuf_v, acc.at[pl.ds(sid * rows_per_tile + i * zrows, zrows)]
            )
        plsc.subcore_barrier()

        def start(i):
            sb = wid + NW * i
            b = i % DEPTH
            dr = pltpu.async_copy(
                x_hbm.at[pl.ds(sb * SB, SB)], rows_v.at[b], sems[b]
            )
            di = pltpu.async_copy(b_hbm.at[pl.ds(sb, 1)], ids_v.at[b], sems[b])
            return dr, di

        def scatter(i):
            b = i % DEPTH
            return pltpu.async_copy(
                rows_v.at[b], acc.at[ids_v.at[b, 0]], ssems[b], add=True
            )

        descs = {0: start(0), 1: start(1)}
        sdescs = {}
        for i in range(unif):
            if i >= 2:
                sdescs.pop(i - 2).wait()
            if i + 2 < unif:
                descs[i + 2] = start(i + 2)
            dr, di = descs.pop(i)
            dr.wait()
            di.wait()
            sdescs[i] = scatter(i)

        for i in sorted(sdescs):
            sdescs.pop(i).wait()

        @pl.when(wid < tail_n)
        def _tail():
            sb = wid + NW * unif
            b = unif % DEPTH
            pltpu.sync_copy(x_hbm.at[pl.ds(sb * SB, SB)], rows_v.at[b])
            pltpu.sync_copy(b_hbm.at[pl.ds(sb, 1)], ids_v.at[b])
            pltpu.sync_copy(rows_v.at[b], acc.at[ids_v.at[b, 0]], add=True)

        plsc.subcore_barrier()
        pltpu.sync_copy(
            acc.at[pl.ds(sid * rows_per_tile, rows_per_tile)],
            out_hbm.at[cid, pl.ds(sid * rows_per_tile, rows_per_tile)],
        )

    return sc_kernel(x, batch2d)


def _combine_body(p_ref, o_ref):
    o_ref[...] = p_ref[0] + p_ref[1]


def kernel(x, batch, ptr):
    n, d = x.shape
    g = int(ptr.shape[0]) - 1
    batch2d = batch.astype(jnp.int32).reshape(n // 128, 128)
    partials = _sc_partial_sums(x, batch2d, n, d, g)
    out = pl.pallas_call(
        _combine_body,
        out_shape=jax.ShapeDtypeStruct((g, d), jnp.float32),
    )(partials)
    return out


# speedup vs baseline: 4.5095x; 1.0018x over previous
import functools

import jax
import jax.numpy as jnp
from jax import lax
from jax.experimental import pallas as pl
from jax.experimental.pallas import tpu as pltpu
from jax.experimental.pallas import tpu_sc as plsc

NC = 2
NS = 16
NW = NC * NS

SB = 128
DEPTH = 6


def _sc_partial_sums(x, batch2d, n_rows, d, g):
    num_sb = n_rows // SB
    unif = num_sb // NW
    tail_n = num_sb - NW * unif
    mesh = plsc.VectorSubcoreMesh(
        core_axis_name="c", subcore_axis_name="s", num_cores=NC, num_subcores=NS
    )
    rows_per_tile = g // NS
    zrows = 8

    @functools.partial(
        pl.kernel,
        out_type=jax.ShapeDtypeStruct((NC, g, d), jnp.float32),
        mesh=mesh,
        scratch_types=[
            pltpu.VMEM((DEPTH, 1, 128), jnp.int32),
            pltpu.VMEM((DEPTH, SB, d), jnp.float32),
            pltpu.VMEM((zrows, d), jnp.float32),
            pltpu.VMEM_SHARED((g, d), jnp.float32),
        ]
        + [pltpu.SemaphoreType.DMA] * (2 * DEPTH),
    )
    def sc_kernel(x_hbm, b_hbm, out_hbm, ids_v, rows_v, zbuf_v, acc, *sems_all):
        cid = lax.axis_index("c")
        sid = lax.axis_index("s")
        wid = sid * NC + cid
        sems = sems_all[:DEPTH]
        ssems = sems_all[DEPTH:]

        zeros16 = jnp.zeros((16,), jnp.float32)

        @pl.loop(0, zrows)
        def _zero(i):
            for k in range(d // 16):
                zbuf_v[i, pl.ds(k * 16, 16)] = zeros16

        for i in range(rows_per_tile // zrows):
            pltpu.sync_copy(
                zbuf_v, acc.at[pl.ds(sid * rows_per_tile + i * zrows, zrows)]
            )
        plsc.subcore_barrier()

        def start(i):
            sb = wid + NW * i
            b = i % DEPTH
            dr = pltpu.async_copy(
                x_hbm.at[pl.ds(sb * SB, SB)], rows_v.at[b], sems[b]
            )
            di = pltpu.async_copy(b_hbm.at[pl.ds(sb, 1)], ids_v.at[b], sems[b])
            return dr, di

        def scatter(i):
            b = i % DEPTH
            return pltpu.async_copy(
                rows_v.at[b], acc.at[ids_v.at[b, 0]], ssems[b], add=True
            )

        descs = {0: start(0), 1: start(1)}
        sdescs = {}
        for i in range(unif):
            if i >= 2:
                sdescs.pop(i - 2).wait()
            if i + 2 < unif:
                descs[i + 2] = start(i + 2)
            dr, di = descs.pop(i)
            dr.wait()
            di.wait()
            sdescs[i] = scatter(i)

        for i in sorted(sdescs):
            sdescs.pop(i).wait()

        @pl.when(wid < tail_n)
        def _tail():
            sb = wid + NW * unif
            b = unif % DEPTH
            pltpu.sync_copy(x_hbm.at[pl.ds(sb * SB, SB)], rows_v.at[b])
            pltpu.sync_copy(b_hbm.at[pl.ds(sb, 1)], ids_v.at[b])
            pltpu.sync_copy(rows_v.at[b], acc.at[ids_v.at[b, 0]], add=True)

        plsc.subcore_barrier()
        pltpu.sync_copy(
            acc.at[pl.ds(sid * rows_per_tile, rows_per_tile)],
            out_hbm.at[cid, pl.ds(sid * rows_per_tile, rows_per_tile)],
        )

    return sc_kernel(x, batch2d)


def _combine_body(p_ref, o_ref):
    o_ref[...] = p_ref[0] + p_ref[1]


def kernel(x, batch, ptr):
    n, d = x.shape
    g = int(ptr.shape[0]) - 1
    batch2d = batch.astype(jnp.int32).reshape(n // 128, 128)
    partials = _sc_partial_sums(x, batch2d, n, d, g)
    out = pl.pallas_call(
        _combine_body,
        out_shape=jax.ShapeDtypeStruct((g, d), jnp.float32),
    )(partials)
    return out
